# baseline (speedup 1.0000x reference)
import functools

import jax
import jax.numpy as jnp
from jax import lax
from jax.experimental import pallas as pl
from jax.experimental.pallas import tpu as pltpu
from jax.experimental.pallas import tpu_sc as plsc

N_SO = 100000
DIM = 64
BATCH = 16384
HIST = 50

NC = 2
NS = 16
NW = NC * NS
LANES = 16

ROWS_PER_W = BATCH // NW
IDX_PER_W = ROWS_PER_W * HIST
BB = 16
NBLK = ROWS_PER_W // BB
IDX_PER_BLK = BB * HIST


TBLK = 512
TGRID = -(-N_SO // TBLK)


def _fmt_body(et_ref, me_ref, mo_ref, out_ref):
    x = et_ref[...].astype(jnp.bfloat16)
    dn = (((1,), (1,)), ((), ()))
    y_even = lax.dot_general(me_ref[...], x, dn,
                             preferred_element_type=jnp.float32)
    y_odd = lax.dot_general(mo_ref[...], x, dn,
                            preferred_element_type=jnp.float32)
    y = jnp.concatenate([y_even, y_odd], axis=1)
    out_ref[...] = y.astype(jnp.bfloat16)


_format_table = pl.pallas_call(
    _fmt_body,
    out_shape=jax.ShapeDtypeStruct((N_SO // 2, 2 * DIM), jnp.bfloat16),
    grid=(TGRID,),
    in_specs=[
        pl.BlockSpec((DIM, TBLK), lambda i: (0, i)),
        pl.BlockSpec((TBLK // 2, TBLK), lambda i: (0, 0)),
        pl.BlockSpec((TBLK // 2, TBLK), lambda i: (0, 0)),
    ],
    out_specs=pl.BlockSpec((TBLK // 2, 2 * DIM), lambda i: (i, 0)),
)


def _sel_masks():
    p = jnp.arange(TBLK // 2, dtype=jnp.int32)[:, None]
    k = jnp.arange(TBLK, dtype=jnp.int32)[None, :]
    return ((k == 2 * p).astype(jnp.bfloat16),
            (k == 2 * p + 1).astype(jnp.bfloat16))


@functools.partial(
    pl.kernel,
    out_type=jax.ShapeDtypeStruct((BATCH, DIM), jnp.float32),
    mesh=plsc.VectorSubcoreMesh(core_axis_name="c", subcore_axis_name="s"),
    scratch_types=[
        pltpu.VMEM((IDX_PER_W,), jnp.int32),
        pltpu.VMEM((IDX_PER_BLK, DIM), jnp.bfloat16),
        pltpu.VMEM((IDX_PER_BLK, DIM), jnp.bfloat16),
        pltpu.VMEM((DIM,), jnp.float32),
        pltpu.VMEM((ROWS_PER_W, DIM), jnp.float32),
        pltpu.SemaphoreType.DMA,
        pltpu.SemaphoreType.DMA,
    ],
    compiler_params=pltpu.CompilerParams(
        use_tc_tiling_on_sc=False, needs_layout_passes=False
    ),
)
def _sc_pool(occ_hbm, e_hbm, b_hbm, out_hbm,
             idx_all, rows_a, rows_b, bias_v, out_all, sem_a, sem_b):
    wid = lax.axis_index("s") * NC + lax.axis_index("c")
    base_row = wid * ROWS_PER_W
    e2 = e_hbm

    pltpu.sync_copy(
        occ_hbm.at[pl.ds(pl.multiple_of(base_row * HIST, IDX_PER_W), IDX_PER_W)],
        idx_all,
    )
    pltpu.sync_copy(b_hbm, bias_v)

    iota = lax.iota(jnp.int32, LANES)
    cols = [(j * 2 * LANES + 2 * iota, j * 2 * LANES + 2 * iota + 1)
            for j in range(2)]
    bias = [(plsc.load_gather(bias_v, [ce]), plsc.load_gather(bias_v, [co]))
            for (ce, co) in cols]

    def start(g, rows_v, sem):
        idx_slice = idx_all.at[pl.ds(g * IDX_PER_BLK, IDX_PER_BLK)]
        pltpu.async_copy(e2.at[idx_slice], rows_v, sem)

    def compute(g, rows_v, sem):
        pltpu.make_async_copy(
            e2.at[idx_all.at[pl.ds(g * IDX_PER_BLK, IDX_PER_BLK)]],
            rows_v, sem,
        ).wait()
        row0 = g * BB

        def row_body(r, carry2):
            rbase = r * HIST

            def h_body(h, accs):
                row = rbase + h
                (a0, b0), (a1, b1) = accs
                v0 = rows_v[row, pl.ds(0, 2 * LANES)]
                e0, o0 = plsc.unpack(v0, format=plsc.PackFormat.INTERLEAVED)
                v1 = rows_v[row, pl.ds(2 * LANES, 2 * LANES)]
                e1, o1 = plsc.unpack(v1, format=plsc.PackFormat.INTERLEAVED)
                return ((a0 + e0, b0 + o0), (a1 + e1, b1 + o1))

            accs = lax.fori_loop(0, HIST, h_body, tuple(bias), unroll=10)
            rvec = jnp.broadcast_to(row0 + r, (LANES,)).astype(jnp.int32)
            for j in range(2):
                ae, ao = accs[j]
                ce, co = cols[j]
                plsc.store_scatter(out_all, [rvec, ce], ae)
                plsc.store_scatter(out_all, [rvec, co], ao)
            return carry2

        lax.fori_loop(0, BB, row_body, 0)

    start(0, rows_a, sem_a)

    def blk_pair(t, carry):
        g0 = t * 2
        g1 = g0 + 1
        start(g1, rows_b, sem_b)
        compute(g0, rows_a, sem_a)

        @pl.when(g1 + 1 < NBLK)
        def _():
            start(g1 + 1, rows_a, sem_a)

        compute(g1, rows_b, sem_b)
        return carry

    lax.fori_loop(0, NBLK // 2, blk_pair, 0)

    pltpu.sync_copy(
        out_all,
        out_hbm.at[pl.ds(pl.multiple_of(base_row, ROWS_PER_W), ROWS_PER_W)],
    )


def kernel(occ_so, E, b):
    m_even, m_odd = _sel_masks()
    e_bf = _format_table(E.T, m_even, m_odd).reshape(N_SO, DIM)
    occ_flat = occ_so.reshape(-1)
    return _sc_pool(occ_flat, e_bf, b)


# speedup vs baseline: 1.0248x; 1.0248x over previous
import functools

import jax
import jax.numpy as jnp
from jax import lax
from jax.experimental import pallas as pl
from jax.experimental.pallas import tpu as pltpu
from jax.experimental.pallas import tpu_sc as plsc

N_SO = 100000
DIM = 64
BATCH = 16384
HIST = 50

NC = 2
NS = 16
NW = NC * NS
LANES = 16

ROWS_PER_W = BATCH // NW
IDX_PER_W = ROWS_PER_W * HIST
BB = 16
NBLK = ROWS_PER_W // BB
IDX_PER_BLK = BB * HIST


TBLK = 2048
TGRID = -(-N_SO // TBLK)


def _fmt_body(et_ref, me_ref, mo_ref, out_ref):
    x = et_ref[...].astype(jnp.bfloat16)
    dn = (((1,), (1,)), ((), ()))
    y_even = lax.dot_general(me_ref[...], x, dn,
                             preferred_element_type=jnp.float32)
    y_odd = lax.dot_general(mo_ref[...], x, dn,
                            preferred_element_type=jnp.float32)
    y = jnp.concatenate([y_even, y_odd], axis=1)
    out_ref[...] = y.astype(jnp.bfloat16)


_format_table = pl.pallas_call(
    _fmt_body,
    out_shape=jax.ShapeDtypeStruct((N_SO // 2, 2 * DIM), jnp.bfloat16),
    grid=(TGRID,),
    in_specs=[
        pl.BlockSpec((DIM, TBLK), lambda i: (0, i)),
        pl.BlockSpec((TBLK // 2, TBLK), lambda i: (0, 0)),
        pl.BlockSpec((TBLK // 2, TBLK), lambda i: (0, 0)),
    ],
    out_specs=pl.BlockSpec((TBLK // 2, 2 * DIM), lambda i: (i, 0)),
)


def _sel_masks():
    p = jnp.arange(TBLK // 2, dtype=jnp.int32)[:, None]
    k = jnp.arange(TBLK, dtype=jnp.int32)[None, :]
    return ((k == 2 * p).astype(jnp.bfloat16),
            (k == 2 * p + 1).astype(jnp.bfloat16))


@functools.partial(
    pl.kernel,
    out_type=jax.ShapeDtypeStruct((BATCH, DIM), jnp.float32),
    mesh=plsc.VectorSubcoreMesh(core_axis_name="c", subcore_axis_name="s"),
    scratch_types=[
        pltpu.VMEM((IDX_PER_W,), jnp.int32),
        pltpu.VMEM((IDX_PER_BLK, DIM), jnp.bfloat16),
        pltpu.VMEM((IDX_PER_BLK, DIM), jnp.bfloat16),
        pltpu.VMEM((DIM,), jnp.float32),
        pltpu.VMEM((ROWS_PER_W, DIM), jnp.float32),
        pltpu.SemaphoreType.DMA,
        pltpu.SemaphoreType.DMA,
    ],
    compiler_params=pltpu.CompilerParams(
        use_tc_tiling_on_sc=False, needs_layout_passes=False
    ),
)
def _sc_pool(occ_hbm, e_hbm, b_hbm, out_hbm,
             idx_all, rows_a, rows_b, bias_v, out_all, sem_a, sem_b):
    wid = lax.axis_index("s") * NC + lax.axis_index("c")
    base_row = wid * ROWS_PER_W
    e2 = e_hbm

    pltpu.sync_copy(
        occ_hbm.at[pl.ds(pl.multiple_of(base_row * HIST, IDX_PER_W), IDX_PER_W)],
        idx_all,
    )
    pltpu.sync_copy(b_hbm, bias_v)

    iota = lax.iota(jnp.int32, LANES)
    cols = [(j * 2 * LANES + 2 * iota, j * 2 * LANES + 2 * iota + 1)
            for j in range(2)]
    bias = [(plsc.load_gather(bias_v, [ce]), plsc.load_gather(bias_v, [co]))
            for (ce, co) in cols]

    def start(g, rows_v, sem):
        idx_slice = idx_all.at[pl.ds(g * IDX_PER_BLK, IDX_PER_BLK)]
        pltpu.async_copy(e2.at[idx_slice], rows_v, sem)

    def compute(g, rows_v, sem):
        pltpu.make_async_copy(
            e2.at[idx_all.at[pl.ds(g * IDX_PER_BLK, IDX_PER_BLK)]],
            rows_v, sem,
        ).wait()
        row0 = g * BB

        def row_body(r, carry2):
            rbase = r * HIST

            def h_body(h, accs):
                row = rbase + h
                (a0, b0), (a1, b1) = accs
                v0 = rows_v[row, pl.ds(0, 2 * LANES)]
                e0, o0 = plsc.unpack(v0, format=plsc.PackFormat.INTERLEAVED)
                v1 = rows_v[row, pl.ds(2 * LANES, 2 * LANES)]
                e1, o1 = plsc.unpack(v1, format=plsc.PackFormat.INTERLEAVED)
                return ((a0 + e0, b0 + o0), (a1 + e1, b1 + o1))

            accs = lax.fori_loop(0, HIST, h_body, tuple(bias), unroll=10)
            rvec = jnp.broadcast_to(row0 + r, (LANES,)).astype(jnp.int32)
            for j in range(2):
                ae, ao = accs[j]
                ce, co = cols[j]
                plsc.store_scatter(out_all, [rvec, ce], ae)
                plsc.store_scatter(out_all, [rvec, co], ao)
            return carry2

        lax.fori_loop(0, BB, row_body, 0)

    start(0, rows_a, sem_a)

    def blk_pair(t, carry):
        g0 = t * 2
        g1 = g0 + 1
        start(g1, rows_b, sem_b)
        compute(g0, rows_a, sem_a)

        @pl.when(g1 + 1 < NBLK)
        def _():
            start(g1 + 1, rows_a, sem_a)

        compute(g1, rows_b, sem_b)
        return carry

    lax.fori_loop(0, NBLK // 2, blk_pair, 0)

    pltpu.sync_copy(
        out_all,
        out_hbm.at[pl.ds(pl.multiple_of(base_row, ROWS_PER_W), ROWS_PER_W)],
    )


def kernel(occ_so, E, b):
    m_even, m_odd = _sel_masks()
    e_bf = _format_table(E.T, m_even, m_odd).reshape(N_SO, DIM)
    occ_flat = occ_so.reshape(-1)
    return _sc_pool(occ_flat, e_bf, b)


# speedup vs baseline: 1.3994x; 1.3655x over previous
import functools

import jax
import jax.numpy as jnp
from jax import lax
from jax.experimental import pallas as pl
from jax.experimental.pallas import tpu as pltpu
from jax.experimental.pallas import tpu_sc as plsc

N_SO = 100000
DIM = 64
BATCH = 16384
HIST = 50

NC = 2
NS = 16
NW = NC * NS
LANES = 16

ROWS_PER_W = BATCH // NW
IDX_PER_W = ROWS_PER_W * HIST
BB = 16
NBLK = ROWS_PER_W // BB
IDX_PER_BLK = BB * HIST


TBLK = 2048
TGRID = -(-N_SO // TBLK)


def _fmt_body(et_ref, out_ref):
    x = et_ref[...].astype(jnp.bfloat16)
    out_ref[...] = jnp.swapaxes(x, 0, 1)


_format_table = pl.pallas_call(
    _fmt_body,
    out_shape=jax.ShapeDtypeStruct((N_SO, DIM), jnp.bfloat16),
    grid=(TGRID,),
    in_specs=[pl.BlockSpec((DIM, TBLK), lambda i: (0, i))],
    out_specs=pl.BlockSpec((TBLK, DIM), lambda i: (i, 0)),
)


def _sel_masks():
    p = jnp.arange(TBLK // 2, dtype=jnp.int32)[:, None]
    k = jnp.arange(TBLK, dtype=jnp.int32)[None, :]
    return ((k == 2 * p).astype(jnp.bfloat16),
            (k == 2 * p + 1).astype(jnp.bfloat16))


@functools.partial(
    pl.kernel,
    out_type=jax.ShapeDtypeStruct((BATCH, DIM), jnp.float32),
    mesh=plsc.VectorSubcoreMesh(core_axis_name="c", subcore_axis_name="s"),
    scratch_types=[
        pltpu.VMEM((IDX_PER_W,), jnp.int32),
        pltpu.VMEM((IDX_PER_BLK, DIM), jnp.bfloat16),
        pltpu.VMEM((IDX_PER_BLK, DIM), jnp.bfloat16),
        pltpu.VMEM((DIM,), jnp.float32),
        pltpu.VMEM((ROWS_PER_W, DIM), jnp.float32),
        pltpu.SemaphoreType.DMA,
        pltpu.SemaphoreType.DMA,
    ],
    compiler_params=pltpu.CompilerParams(
        use_tc_tiling_on_sc=False, needs_layout_passes=False
    ),
)
def _sc_pool(occ_hbm, e_hbm, b_hbm, out_hbm,
             idx_all, rows_a, rows_b, bias_v, out_all, sem_a, sem_b):
    wid = lax.axis_index("s") * NC + lax.axis_index("c")
    base_row = wid * ROWS_PER_W
    e2 = e_hbm

    pltpu.sync_copy(
        occ_hbm.at[pl.ds(pl.multiple_of(base_row * HIST, IDX_PER_W), IDX_PER_W)],
        idx_all,
    )
    pltpu.sync_copy(b_hbm, bias_v)

    iota = lax.iota(jnp.int32, LANES)
    cols = [(j * 2 * LANES + 2 * iota, j * 2 * LANES + 2 * iota + 1)
            for j in range(2)]
    bias = [(plsc.load_gather(bias_v, [ce]), plsc.load_gather(bias_v, [co]))
            for (ce, co) in cols]

    def start(g, rows_v, sem):
        idx_slice = idx_all.at[pl.ds(g * IDX_PER_BLK, IDX_PER_BLK)]
        pltpu.async_copy(e2.at[idx_slice], rows_v, sem)

    def compute(g, rows_v, sem):
        pltpu.make_async_copy(
            e2.at[idx_all.at[pl.ds(g * IDX_PER_BLK, IDX_PER_BLK)]],
            rows_v, sem,
        ).wait()
        row0 = g * BB

        def row_body(r, carry2):
            rbase = r * HIST

            def h_body(h, accs):
                row = rbase + h
                (a0, b0), (a1, b1) = accs
                v0 = rows_v[row, pl.ds(0, 2 * LANES)]
                e0, o0 = plsc.unpack(v0, format=plsc.PackFormat.INTERLEAVED)
                v1 = rows_v[row, pl.ds(2 * LANES, 2 * LANES)]
                e1, o1 = plsc.unpack(v1, format=plsc.PackFormat.INTERLEAVED)
                return ((a0 + e0, b0 + o0), (a1 + e1, b1 + o1))

            accs = lax.fori_loop(0, HIST, h_body, tuple(bias), unroll=10)
            rvec = jnp.broadcast_to(row0 + r, (LANES,)).astype(jnp.int32)
            for j in range(2):
                ae, ao = accs[j]
                ce, co = cols[j]
                plsc.store_scatter(out_all, [rvec, ce], ae)
                plsc.store_scatter(out_all, [rvec, co], ao)
            return carry2

        lax.fori_loop(0, BB, row_body, 0)

    start(0, rows_a, sem_a)

    def blk_pair(t, carry):
        g0 = t * 2
        g1 = g0 + 1
        start(g1, rows_b, sem_b)
        compute(g0, rows_a, sem_a)

        @pl.when(g1 + 1 < NBLK)
        def _():
            start(g1 + 1, rows_a, sem_a)

        compute(g1, rows_b, sem_b)
        return carry

    lax.fori_loop(0, NBLK // 2, blk_pair, 0)

    pltpu.sync_copy(
        out_all,
        out_hbm.at[pl.ds(pl.multiple_of(base_row, ROWS_PER_W), ROWS_PER_W)],
    )


def kernel(occ_so, E, b):
    e_bf = _format_table(E.T)
    occ_flat = occ_so.reshape(-1)
    return _sc_pool(occ_flat, e_bf, b)


# speedup vs baseline: 1.4923x; 1.0664x over previous
import functools

import jax
import jax.numpy as jnp
from jax import lax
from jax.experimental import pallas as pl
from jax.experimental.pallas import tpu as pltpu
from jax.experimental.pallas import tpu_sc as plsc

N_SO = 100000
DIM = 64
BATCH = 16384
HIST = 50

NC = 2
NS = 16
NW = NC * NS
LANES = 16

ROWS_PER_W = BATCH // NW
IDX_PER_W = ROWS_PER_W * HIST
BB = 16
NBLK = ROWS_PER_W // BB
IDX_PER_BLK = BB * HIST


@functools.partial(
    pl.kernel,
    out_type=jax.ShapeDtypeStruct((BATCH, DIM), jnp.float32),
    mesh=plsc.VectorSubcoreMesh(core_axis_name="c", subcore_axis_name="s"),
    scratch_types=[
        pltpu.VMEM((IDX_PER_W,), jnp.int32),
        pltpu.VMEM((IDX_PER_BLK, DIM), jnp.bfloat16),
        pltpu.VMEM((IDX_PER_BLK, DIM), jnp.bfloat16),
        pltpu.VMEM((DIM,), jnp.float32),
        pltpu.VMEM((ROWS_PER_W, DIM), jnp.float32),
        pltpu.SemaphoreType.DMA,
        pltpu.SemaphoreType.DMA,
    ],
    compiler_params=pltpu.CompilerParams(
        use_tc_tiling_on_sc=False, needs_layout_passes=False
    ),
)
def _sc_pool(occ_hbm, e_hbm, b_hbm, out_hbm,
             idx_all, rows_a, rows_b, bias_v, out_all, sem_a, sem_b):
    wid = lax.axis_index("s") * NC + lax.axis_index("c")
    base_row = wid * ROWS_PER_W

    pltpu.sync_copy(
        occ_hbm.at[pl.ds(pl.multiple_of(base_row * HIST, IDX_PER_W), IDX_PER_W)],
        idx_all,
    )
    pltpu.sync_copy(b_hbm, bias_v)

    iota = lax.iota(jnp.int32, LANES)
    cols = [(j * 2 * LANES + 2 * iota, j * 2 * LANES + 2 * iota + 1)
            for j in range(2)]
    bias = [(plsc.load_gather(bias_v, [ce]), plsc.load_gather(bias_v, [co]))
            for (ce, co) in cols]

    def start(g, rows_v, sem):
        idx_slice = idx_all.at[pl.ds(g * IDX_PER_BLK, IDX_PER_BLK)]
        pltpu.async_copy(e_hbm.at[idx_slice], rows_v, sem)

    def compute(g, rows_v, sem):
        pltpu.make_async_copy(
            e_hbm.at[idx_all.at[pl.ds(g * IDX_PER_BLK, IDX_PER_BLK)]],
            rows_v, sem,
        ).wait()
        row0 = g * BB

        def row_body(r, carry2):
            rbase = r * HIST

            def h_body(h, accs):
                row = rbase + h
                (a0, b0), (a1, b1) = accs
                v0 = rows_v[row, pl.ds(0, 2 * LANES)]
                e0, o0 = plsc.unpack(v0, format=plsc.PackFormat.INTERLEAVED)
                v1 = rows_v[row, pl.ds(2 * LANES, 2 * LANES)]
                e1, o1 = plsc.unpack(v1, format=plsc.PackFormat.INTERLEAVED)
                return ((a0 + e0, b0 + o0), (a1 + e1, b1 + o1))

            accs = lax.fori_loop(0, HIST, h_body, tuple(bias), unroll=10)
            rvec = jnp.broadcast_to(row0 + r, (LANES,)).astype(jnp.int32)
            for j in range(2):
                ae, ao = accs[j]
                ce, co = cols[j]
                plsc.store_scatter(out_all, [rvec, ce], ae)
                plsc.store_scatter(out_all, [rvec, co], ao)
            return carry2

        lax.fori_loop(0, BB, row_body, 0)

    start(0, rows_a, sem_a)

    def blk_pair(t, carry):
        g0 = t * 2
        g1 = g0 + 1
        start(g1, rows_b, sem_b)
        compute(g0, rows_a, sem_a)

        @pl.when(g1 + 1 < NBLK)
        def _():
            start(g1 + 1, rows_a, sem_a)

        compute(g1, rows_b, sem_b)
        return carry

    lax.fori_loop(0, NBLK // 2, blk_pair, 0)

    pltpu.sync_copy(
        out_all,
        out_hbm.at[pl.ds(pl.multiple_of(base_row, ROWS_PER_W), ROWS_PER_W)],
    )


def kernel(occ_so, E, b):
    e_bf = lax.optimization_barrier(E.T.astype(jnp.bfloat16)).T
    occ_flat = occ_so.reshape(-1)
    return _sc_pool(occ_flat, e_bf, b)
